# baseline (speedup 1.0000x reference)
import functools

import jax
import jax.numpy as jnp
from jax import lax
from jax.experimental import pallas as pl
from jax.experimental.pallas import tpu as pltpu
from jax.experimental.pallas import tpu_sc as plsc

NC = 2
NS = 16
NW = NC * NS

N_NODES = 10000
D = 128
E_TOTAL = 320000
E_PER_W = E_TOTAL // NW
CHUNK = 80
N_CHUNKS = E_PER_W // CHUNK


def _sc_body(xd_hbm, xp_hbm, idd_hbm, idp_hbm, out_hbm,
             idd_v, idp_v, rows_a, rows_b, out_v, sem_i, sem_g0, sem_g1):
  wid = lax.axis_index("s") * NC + lax.axis_index("c")
  base_w = wid * E_PER_W
  lane = lax.iota(jnp.int32, 16)
  sem_g = (sem_g0, sem_g1)

  def issue_idx(k, b):
    base = base_w + k * CHUNK
    pltpu.async_copy(idd_hbm.at[pl.ds(base, CHUNK)], idd_v.at[b], sem_i)
    pltpu.async_copy(idp_hbm.at[pl.ds(base, CHUNK)], idp_v.at[b], sem_i)

  def wait_idx(b):
    pltpu.make_async_copy(
        idd_hbm.at[pl.ds(0, CHUNK)], idd_v.at[b], sem_i).wait()
    pltpu.make_async_copy(
        idp_hbm.at[pl.ds(0, CHUNK)], idp_v.at[b], sem_i).wait()

  def issue_gather(b):
    pltpu.async_copy(xd_hbm.at[idd_v.at[b]], rows_a.at[b], sem_g[b])
    pltpu.async_copy(xp_hbm.at[idp_v.at[b]], rows_b.at[b], sem_g[b])

  def wait_gather(b):
    pltpu.make_async_copy(
        xd_hbm.at[idd_v.at[b]], rows_a.at[b], sem_g[b]).wait()
    pltpu.make_async_copy(
        xp_hbm.at[idp_v.at[b]], rows_b.at[b], sem_g[b]).wait()

  def compute(k, b):
    out_base = k * CHUNK

    def group_body(g, c):
      e0 = g * 16
      res = jnp.zeros((16,), jnp.float32)
      for i in range(16):
        e = e0 + i
        pr = []
        for j in range(D // 32):
          wa = rows_a[b, e, pl.ds(j * 32, 32)]
          wb = rows_b[b, e, pl.ds(j * 32, 32)]
          pr.append(wa * wb)
        s = (pr[0] + pr[1]) + (pr[2] + pr[3])
        t0, t1 = plsc.unpack(s, format=plsc.PackFormat.INTERLEAVED)
        res = jnp.where(lane == i, jnp.sum(t0 + t1), res)
      out_v[pl.ds(out_base + e0, 16)] = res
      return c

    lax.fori_loop(0, CHUNK // 16, group_body, 0)

  pltpu.sync_copy(idd_hbm.at[pl.ds(base_w, CHUNK)], idd_v.at[0])
  pltpu.sync_copy(idp_hbm.at[pl.ds(base_w, CHUNK)], idp_v.at[0])
  issue_gather(0)
  issue_idx(1, 1)

  def phase(k, b):
    @pl.when(k < N_CHUNKS - 1)
    def _():
      wait_idx(b ^ 1)
      issue_gather(b ^ 1)

    wait_gather(b)

    @pl.when(k < N_CHUNKS - 2)
    def _():
      issue_idx(k + 2, b)

    compute(k, b)

  def pair_body(k2, c):
    k = k2 * 2
    phase(k, 0)
    phase(k + 1, 1)
    return c

  lax.fori_loop(0, N_CHUNKS // 2, pair_body, 0)
  if N_CHUNKS % 2:
    phase(N_CHUNKS - 1, 0)

  pltpu.sync_copy(out_v, out_hbm.at[pl.ds(base_w, E_PER_W)])


@functools.partial(jax.jit, static_argnames=("interpret",))
def _run(xd, xp, idd, idp, interpret=False):
  mesh = plsc.VectorSubcoreMesh(core_axis_name="c", subcore_axis_name="s",
                                num_cores=NC, num_subcores=NS)
  return pl.kernel(
      _sc_body,
      out_type=jax.ShapeDtypeStruct((E_TOTAL,), jnp.float32),
      mesh=mesh,
      scratch_types=[
          pltpu.VMEM((2, CHUNK), jnp.int32),
          pltpu.VMEM((2, CHUNK), jnp.int32),
          pltpu.VMEM((2, CHUNK, D), jnp.bfloat16),
          pltpu.VMEM((2, CHUNK, D), jnp.bfloat16),
          pltpu.VMEM((E_PER_W,), jnp.float32),
          pltpu.SemaphoreType.DMA,
          pltpu.SemaphoreType.DMA,
          pltpu.SemaphoreType.DMA,
      ],
      compiler_params=pltpu.CompilerParams(needs_layout_passes=False,
                                           use_tc_tiling_on_sc=False),
      interpret=interpret,
  )(xd, xp, idd, idp)


def kernel(x_drug, x_prot, edge_label_index):
  eli = edge_label_index.astype(jnp.int32)
  return _run(x_drug.astype(jnp.bfloat16), x_prot.astype(jnp.bfloat16),
              eli[0], eli[1])


# speedup vs baseline: 1.2272x; 1.2272x over previous
import functools

import jax
import jax.numpy as jnp
from jax import lax
from jax.experimental import pallas as pl
from jax.experimental.pallas import tpu as pltpu
from jax.experimental.pallas import tpu_sc as plsc

NC = 2
NS = 16
NW = NC * NS

N_NODES = 10000
D = 128
E_TOTAL = 320000
E_PER_W = E_TOTAL // NW
CHUNK = 80
N_CHUNKS = E_PER_W // CHUNK


def _sc_body(xd_hbm, xp_hbm, idd_hbm, idp_hbm, out_hbm,
             idd_v, idp_v, rows_a, rows_b, out_v, sh_a, sh_b,
             sem_i, sem_g0, sem_g1):
  sid = lax.axis_index("s")
  wid = sid * NC + lax.axis_index("c")
  base_w = wid * E_PER_W

  rpt = N_NODES // NS
  t0 = sid * rpt
  pltpu.sync_copy(xd_hbm.at[pl.ds(t0, rpt)], sh_a.at[pl.ds(t0, rpt)])
  pltpu.sync_copy(xp_hbm.at[pl.ds(t0, rpt)], sh_b.at[pl.ds(t0, rpt)])
  plsc.subcore_barrier()
  lane = lax.iota(jnp.int32, 16)
  sem_g = (sem_g0, sem_g1)

  def issue_idx(k, b):
    base = base_w + k * CHUNK
    pltpu.async_copy(idd_hbm.at[pl.ds(base, CHUNK)], idd_v.at[b], sem_i)
    pltpu.async_copy(idp_hbm.at[pl.ds(base, CHUNK)], idp_v.at[b], sem_i)

  def wait_idx(b):
    pltpu.make_async_copy(
        idd_hbm.at[pl.ds(0, CHUNK)], idd_v.at[b], sem_i).wait()
    pltpu.make_async_copy(
        idp_hbm.at[pl.ds(0, CHUNK)], idp_v.at[b], sem_i).wait()

  def issue_gather(b):
    pltpu.async_copy(sh_a.at[idd_v.at[b]], rows_a.at[b], sem_g[b])
    pltpu.async_copy(sh_b.at[idp_v.at[b]], rows_b.at[b], sem_g[b])

  def wait_gather(b):
    pltpu.make_async_copy(
        sh_a.at[idd_v.at[b]], rows_a.at[b], sem_g[b]).wait()
    pltpu.make_async_copy(
        sh_b.at[idp_v.at[b]], rows_b.at[b], sem_g[b]).wait()

  def compute(k, b):
    out_base = k * CHUNK

    def group_body(g, c):
      e0 = g * 16
      res = jnp.zeros((16,), jnp.float32)
      for i in range(16):
        e = e0 + i
        pr = []
        for j in range(D // 32):
          wa = rows_a[b, e, pl.ds(j * 32, 32)]
          wb = rows_b[b, e, pl.ds(j * 32, 32)]
          pr.append(wa * wb)
        s = (pr[0] + pr[1]) + (pr[2] + pr[3])
        t0, t1 = plsc.unpack(s, format=plsc.PackFormat.INTERLEAVED)
        res = jnp.where(lane == i, jnp.sum(t0 + t1), res)
      out_v[pl.ds(out_base + e0, 16)] = res
      return c

    lax.fori_loop(0, CHUNK // 16, group_body, 0)

  pltpu.sync_copy(idd_hbm.at[pl.ds(base_w, CHUNK)], idd_v.at[0])
  pltpu.sync_copy(idp_hbm.at[pl.ds(base_w, CHUNK)], idp_v.at[0])
  issue_gather(0)
  issue_idx(1, 1)

  def phase(k, b):
    @pl.when(k < N_CHUNKS - 1)
    def _():
      wait_idx(b ^ 1)
      issue_gather(b ^ 1)

    wait_gather(b)

    @pl.when(k < N_CHUNKS - 2)
    def _():
      issue_idx(k + 2, b)

    compute(k, b)

  def pair_body(k2, c):
    k = k2 * 2
    phase(k, 0)
    phase(k + 1, 1)
    return c

  lax.fori_loop(0, N_CHUNKS // 2, pair_body, 0)
  if N_CHUNKS % 2:
    phase(N_CHUNKS - 1, 0)

  pltpu.sync_copy(out_v, out_hbm.at[pl.ds(base_w, E_PER_W)])


@functools.partial(jax.jit, static_argnames=("interpret",))
def _run(xd, xp, idd, idp, interpret=False):
  mesh = plsc.VectorSubcoreMesh(core_axis_name="c", subcore_axis_name="s",
                                num_cores=NC, num_subcores=NS)
  return pl.kernel(
      _sc_body,
      out_type=jax.ShapeDtypeStruct((E_TOTAL,), jnp.float32),
      mesh=mesh,
      scratch_types=[
          pltpu.VMEM((2, CHUNK), jnp.int32),
          pltpu.VMEM((2, CHUNK), jnp.int32),
          pltpu.VMEM((2, CHUNK, D), jnp.bfloat16),
          pltpu.VMEM((2, CHUNK, D), jnp.bfloat16),
          pltpu.VMEM((E_PER_W,), jnp.float32),
          pltpu.VMEM_SHARED((N_NODES, D), jnp.bfloat16),
          pltpu.VMEM_SHARED((N_NODES, D), jnp.bfloat16),
          pltpu.SemaphoreType.DMA,
          pltpu.SemaphoreType.DMA,
          pltpu.SemaphoreType.DMA,
      ],
      compiler_params=pltpu.CompilerParams(needs_layout_passes=False,
                                           use_tc_tiling_on_sc=False),
      interpret=interpret,
  )(xd, xp, idd, idp)


def kernel(x_drug, x_prot, edge_label_index):
  eli = edge_label_index.astype(jnp.int32)
  return _run(x_drug.astype(jnp.bfloat16), x_prot.astype(jnp.bfloat16),
              eli[0], eli[1])
